# baseline (speedup 1.0000x reference)
import functools

import jax
import jax.numpy as jnp
from jax import lax
from jax.experimental import pallas as pl
from jax.experimental.pallas import tpu as pltpu
from jax.experimental.pallas import tpu_sc as plsc

VOCAB = 100000
EMBED_DIM = 64
NUM_CLASSES = 4
BATCH = 16384
SEQ = 200

LANES = 16
NUM_CORES = 2
NUM_SUBCORES = 16
NUM_WORKERS = NUM_CORES * NUM_SUBCORES
ROWS_PER_W = BATCH // NUM_WORKERS
GROUP = 16
NGROUPS = ROWS_PER_W // GROUP
IDX_PER_GROUP = GROUP * SEQ
STREAM_LEN = 128
NSTREAMS = IDX_PER_GROUP // STREAM_LEN

_PROJ_BLOCK = 10000


def _project_body(e_ref, w_ref, b_ref, o_ref):
    o_ref[...] = (
        jnp.dot(e_ref[...], w_ref[...], preferred_element_type=jnp.float32)
        + b_ref[...]
    )


def _project(emb_table, w4, b4):
    grid = (VOCAB // _PROJ_BLOCK,)
    return pl.pallas_call(
        _project_body,
        grid=grid,
        in_specs=[
            pl.BlockSpec((_PROJ_BLOCK, EMBED_DIM), lambda i: (i, 0)),
            pl.BlockSpec((EMBED_DIM, NUM_CLASSES), lambda i: (0, 0)),
            pl.BlockSpec((1, NUM_CLASSES), lambda i: (0, 0)),
        ],
        out_specs=pl.BlockSpec((_PROJ_BLOCK, NUM_CLASSES), lambda i: (i, 0)),
        out_shape=jax.ShapeDtypeStruct((VOCAB, NUM_CLASSES), jnp.float32),
    )(emb_table, w4, b4)


_sc_mesh = plsc.VectorSubcoreMesh(core_axis_name="c", subcore_axis_name="s")


@functools.partial(
    pl.kernel,
    mesh=_sc_mesh,
    compiler_params=pltpu.CompilerParams(
        use_tc_tiling_on_sc=False, needs_layout_passes=False
    ),
    out_type=jax.ShapeDtypeStruct((BATCH * NUM_CLASSES,), jnp.float32),
    scratch_types=[
        pltpu.VMEM((IDX_PER_GROUP,), jnp.int32),
        pltpu.VMEM((IDX_PER_GROUP, NUM_CLASSES), jnp.float32),
        pltpu.VMEM((ROWS_PER_W * NUM_CLASSES,), jnp.float32),
        pltpu.VMEM((LANES,), jnp.float32),
        pltpu.SemaphoreType.DMA,
    ],
)
def _pool_kernel(text_hbm, p_hbm, out_hbm, idx_v, rows_v, out_v, fold_v, sem):
    wid = lax.axis_index("s") * NUM_CORES + lax.axis_index("c")
    tok_base = wid * (ROWS_PER_W * SEQ)

    lane = lax.iota(jnp.int32, LANES)
    m0 = lane < 4
    m1 = lane < 8
    m2 = lane < 12
    row_pat = lane >> 2
    col_pat = lane & 3
    perm8 = (lane + 8) & 15
    perm4 = (lane + 4) & 15

    def row_sum(row0):
        def chunk(kb, accs):
            a0, a1 = accs
            b = row0 + kb * 40
            for u in range(5):
                r0 = row_pat + (b + 8 * u)
                r1 = row_pat + (b + 8 * u + 4)
                a0 = a0 + plsc.load_gather(rows_v, [r0, col_pat])
                a1 = a1 + plsc.load_gather(rows_v, [r1, col_pat])
            return (a0, a1)

        zero = jnp.zeros((LANES,), jnp.float32)
        a0, a1 = lax.fori_loop(0, SEQ // 40, chunk, (zero, zero))
        acc = a0 + a1
        fold_v[...] = acc
        acc = acc + plsc.load_gather(fold_v, [perm8])
        fold_v[...] = acc
        acc = acc + plsc.load_gather(fold_v, [perm4])
        return acc

    def group_body(g, carry):
        off = tok_base + g * IDX_PER_GROUP
        pltpu.sync_copy(text_hbm.at[pl.ds(off, IDX_PER_GROUP)], idx_v)

        def fire(j, c):
            pltpu.async_copy(
                p_hbm.at[idx_v.at[pl.ds(j * STREAM_LEN, STREAM_LEN)]],
                rows_v.at[pl.ds(j * STREAM_LEN, STREAM_LEN)],
                sem,
            )
            return c

        lax.fori_loop(0, NSTREAMS, fire, 0)

        def drain(j, c):
            pltpu.make_async_copy(
                p_hbm.at[idx_v.at[pl.ds(j * STREAM_LEN, STREAM_LEN)]],
                rows_v.at[pl.ds(j * STREAM_LEN, STREAM_LEN)],
                sem,
            ).wait()
            return c

        lax.fori_loop(0, NSTREAMS, drain, 0)

        def quad_body(q, c):
            accs = [row_sum((q * 4 + rr) * SEQ) for rr in range(4)]
            out16 = jnp.where(m0, accs[0],
                              jnp.where(m1, accs[1],
                                        jnp.where(m2, accs[2], accs[3])))
            out_v[pl.ds(g * (GROUP * NUM_CLASSES) + q * LANES, LANES)] = out16
            return c

        lax.fori_loop(0, GROUP // 4, quad_body, 0)
        return carry

    lax.fori_loop(0, NGROUPS, group_body, 0)
    pltpu.sync_copy(
        out_v,
        out_hbm.at[pl.ds(wid * (ROWS_PER_W * NUM_CLASSES),
                         ROWS_PER_W * NUM_CLASSES)],
    )


def kernel(x_batch, emb_table, fc_w, fc_b):
    text = x_batch[:, 1:].astype(jnp.int32).reshape(-1)
    scale = jnp.float32(1.0 / SEQ)
    w4 = fc_w.T * scale
    b4 = (fc_b * scale).reshape(1, NUM_CLASSES)
    p4 = _project(emb_table, w4, b4)
    out = _pool_kernel(text, p4)
    return out.reshape(BATCH, NUM_CLASSES)


# speedup vs baseline: 1.3119x; 1.3119x over previous
import functools

import jax
import jax.numpy as jnp
from jax import lax
from jax.experimental import pallas as pl
from jax.experimental.pallas import tpu as pltpu
from jax.experimental.pallas import tpu_sc as plsc

VOCAB = 100000
EMBED_DIM = 64
NUM_CLASSES = 4
BATCH = 16384
SEQ = 200

LANES = 16
NUM_CORES = 2
NUM_SUBCORES = 16
NUM_WORKERS = NUM_CORES * NUM_SUBCORES
ROWS_PER_W = BATCH // NUM_WORKERS
GROUP = 16
NGROUPS = ROWS_PER_W // GROUP
IDX_PER_GROUP = GROUP * SEQ
STREAM_LEN = 128
NSTREAMS = IDX_PER_GROUP // STREAM_LEN

_PROJ_BLOCK = 10000


def _project_body(e_ref, w_ref, b_ref, o_ref):
    o_ref[...] = (
        jnp.dot(e_ref[...], w_ref[...], preferred_element_type=jnp.float32)
        + b_ref[...]
    )


def _project(emb_table, w16, b16):
    grid = (VOCAB // _PROJ_BLOCK,)
    return pl.pallas_call(
        _project_body,
        grid=grid,
        in_specs=[
            pl.BlockSpec((_PROJ_BLOCK, EMBED_DIM), lambda i: (i, 0)),
            pl.BlockSpec((EMBED_DIM, LANES), lambda i: (0, 0)),
            pl.BlockSpec((1, LANES), lambda i: (0, 0)),
        ],
        out_specs=pl.BlockSpec((_PROJ_BLOCK, LANES), lambda i: (i, 0)),
        out_shape=jax.ShapeDtypeStruct((VOCAB, LANES), jnp.float32),
    )(emb_table, w16, b16)


_sc_mesh = plsc.VectorSubcoreMesh(core_axis_name="c", subcore_axis_name="s")


@functools.partial(
    pl.kernel,
    mesh=_sc_mesh,
    compiler_params=pltpu.CompilerParams(
        use_tc_tiling_on_sc=False, needs_layout_passes=False
    ),
    out_type=jax.ShapeDtypeStruct((BATCH * NUM_CLASSES,), jnp.float32),
    scratch_types=[
        pltpu.VMEM((IDX_PER_GROUP,), jnp.int32),
        pltpu.VMEM((IDX_PER_GROUP,), jnp.int32),
        pltpu.VMEM((IDX_PER_GROUP, LANES), jnp.float32),
        pltpu.VMEM((IDX_PER_GROUP, LANES), jnp.float32),
        pltpu.VMEM((ROWS_PER_W * NUM_CLASSES,), jnp.float32),
        pltpu.VMEM((LANES,), jnp.float32),
        pltpu.SemaphoreType.DMA,
        pltpu.SemaphoreType.DMA,
    ],
)
def _pool_kernel(text_hbm, p_hbm, out_hbm, idx_a, idx_b, rows_a, rows_b,
                 out_v, fold_v, sem_a, sem_b):
    wid = lax.axis_index("s") * NUM_CORES + lax.axis_index("c")
    tok_base = wid * (ROWS_PER_W * SEQ)

    lane = lax.iota(jnp.int32, LANES)
    m0 = lane < 4
    m1 = lane < 8
    m2 = lane < 12
    row_pat = lane >> 2
    col_pat = lane & 3
    perm8 = (lane + 8) & 15
    perm4 = (lane + 4) & 15

    def fire(g, idx_v, rows_v, sem):
        off = tok_base + g * IDX_PER_GROUP
        pltpu.sync_copy(text_hbm.at[pl.ds(off, IDX_PER_GROUP)], idx_v)

        def issue(j, c):
            pltpu.async_copy(
                p_hbm.at[idx_v.at[pl.ds(j * STREAM_LEN, STREAM_LEN)]],
                rows_v.at[pl.ds(j * STREAM_LEN, STREAM_LEN)],
                sem,
            )
            return c

        lax.fori_loop(0, NSTREAMS, issue, 0)

    def drain(idx_v, rows_v, sem):
        def body(j, c):
            pltpu.make_async_copy(
                p_hbm.at[idx_v.at[pl.ds(j * STREAM_LEN, STREAM_LEN)]],
                rows_v.at[pl.ds(j * STREAM_LEN, STREAM_LEN)],
                sem,
            ).wait()
            return c

        lax.fori_loop(0, NSTREAMS, body, 0)

    def accumulate(g, rows_v):
        def row_sum(row0):
            def chunk(kb, accs):
                a0, a1 = accs
                b = row0 + kb * 40
                for u in range(5):
                    r0 = row_pat + (b + 8 * u)
                    r1 = row_pat + (b + 8 * u + 4)
                    a0 = a0 + plsc.load_gather(rows_v, [r0, col_pat])
                    a1 = a1 + plsc.load_gather(rows_v, [r1, col_pat])
                return (a0, a1)

            zero = jnp.zeros((LANES,), jnp.float32)
            a0, a1 = lax.fori_loop(0, SEQ // 40, chunk, (zero, zero))
            acc = a0 + a1
            fold_v[...] = acc
            acc = acc + plsc.load_gather(fold_v, [perm8])
            fold_v[...] = acc
            acc = acc + plsc.load_gather(fold_v, [perm4])
            return acc

        def quad_body(q, c):
            accs = [row_sum((q * 4 + rr) * SEQ) for rr in range(4)]
            out16 = jnp.where(m0, accs[0],
                              jnp.where(m1, accs[1],
                                        jnp.where(m2, accs[2], accs[3])))
            out_v[pl.ds(g * (GROUP * NUM_CLASSES) + q * LANES, LANES)] = out16
            return c

        lax.fori_loop(0, GROUP // 4, quad_body, 0)

    fire(0, idx_a, rows_a, sem_a)

    def pair_body(gp, carry):
        g = gp * 2
        fire(g + 1, idx_b, rows_b, sem_b)
        drain(idx_a, rows_a, sem_a)
        accumulate(g, rows_a)

        @pl.when(gp < NGROUPS // 2 - 1)
        def _():
            fire(g + 2, idx_a, rows_a, sem_a)

        drain(idx_b, rows_b, sem_b)
        accumulate(g + 1, rows_b)
        return carry

    lax.fori_loop(0, NGROUPS // 2, pair_body, 0)
    pltpu.sync_copy(
        out_v,
        out_hbm.at[pl.ds(wid * (ROWS_PER_W * NUM_CLASSES),
                         ROWS_PER_W * NUM_CLASSES)],
    )


def kernel(x_batch, emb_table, fc_w, fc_b):
    text = x_batch[:, 1:].astype(jnp.int32).reshape(-1)
    scale = jnp.float32(1.0 / SEQ)
    w16 = jnp.tile(fc_w.T * scale, (1, 4))
    b16 = jnp.tile(fc_b * scale, 4).reshape(1, LANES)
    p16 = _project(emb_table, w16, b16)
    out = _pool_kernel(text, p16)
    return out.reshape(BATCH, NUM_CLASSES)


# speedup vs baseline: 1.3379x; 1.0198x over previous
import functools

import jax
import jax.numpy as jnp
from jax import lax
from jax.experimental import pallas as pl
from jax.experimental.pallas import tpu as pltpu
from jax.experimental.pallas import tpu_sc as plsc

VOCAB = 100000
EMBED_DIM = 64
NUM_CLASSES = 4
BATCH = 16384
SEQ = 200

LANES = 16
NUM_CORES = 2
NUM_SUBCORES = 16
NUM_WORKERS = NUM_CORES * NUM_SUBCORES
ROWS_PER_W = BATCH // NUM_WORKERS
GROUP = 16
NGROUPS = ROWS_PER_W // GROUP
ROW_PITCH = 208
STREAM_LEN = 128
TAIL_LEN = SEQ + 1 - STREAM_LEN

_PROJ_BLOCK = 10000


def _project_body(e_ref, w_ref, b_ref, o_ref):
    o_ref[...] = (
        jnp.dot(e_ref[...], w_ref[...], preferred_element_type=jnp.float32)
        + b_ref[...]
    )


def _project(emb_table, w16, b16):
    grid = (VOCAB // _PROJ_BLOCK,)
    return pl.pallas_call(
        _project_body,
        grid=grid,
        in_specs=[
            pl.BlockSpec((_PROJ_BLOCK, EMBED_DIM), lambda i: (i, 0)),
            pl.BlockSpec((EMBED_DIM, LANES), lambda i: (0, 0)),
            pl.BlockSpec((1, LANES), lambda i: (0, 0)),
        ],
        out_specs=pl.BlockSpec((_PROJ_BLOCK, LANES), lambda i: (i, 0)),
        out_shape=jax.ShapeDtypeStruct((VOCAB, LANES), jnp.float32),
    )(emb_table, w16, b16)


_sc_mesh = plsc.VectorSubcoreMesh(core_axis_name="c", subcore_axis_name="s")


@functools.partial(
    pl.kernel,
    mesh=_sc_mesh,
    compiler_params=pltpu.CompilerParams(
        use_tc_tiling_on_sc=False, needs_layout_passes=False
    ),
    out_type=jax.ShapeDtypeStruct((BATCH * NUM_CLASSES,), jnp.float32),
    scratch_types=[
        pltpu.VMEM((GROUP, SEQ + 1), jnp.int32),
        pltpu.VMEM((GROUP, SEQ + 1), jnp.int32),
        pltpu.VMEM((GROUP * ROW_PITCH, LANES), jnp.float32),
        pltpu.VMEM((GROUP * ROW_PITCH, LANES), jnp.float32),
        pltpu.VMEM((ROWS_PER_W * NUM_CLASSES,), jnp.float32),
        pltpu.VMEM((LANES,), jnp.float32),
        pltpu.SemaphoreType.DMA,
        pltpu.SemaphoreType.DMA,
    ],
)
def _pool_kernel(text_hbm, p_hbm, out_hbm, idx_a, idx_b, rows_a, rows_b,
                 out_v, fold_v, sem_a, sem_b):
    wid = lax.axis_index("s") * NUM_CORES + lax.axis_index("c")
    row_base = wid * ROWS_PER_W

    lane = lax.iota(jnp.int32, LANES)
    m0 = lane < 4
    m1 = lane < 8
    m2 = lane < 12
    row_pat = lane >> 2
    col_pat = lane & 3
    perm8 = (lane + 8) & 15
    perm4 = (lane + 4) & 15

    def fire(g, idx_v, rows_v, sem):
        pltpu.sync_copy(
            text_hbm.at[pl.ds(row_base + g * GROUP, GROUP)],
            idx_v,
        )

        def issue(r, c):
            pltpu.async_copy(
                p_hbm.at[idx_v.at[r, pl.ds(0, STREAM_LEN)]],
                rows_v.at[pl.ds(r * ROW_PITCH, STREAM_LEN)],
                sem,
            )
            pltpu.async_copy(
                p_hbm.at[idx_v.at[r, pl.ds(STREAM_LEN, TAIL_LEN)]],
                rows_v.at[pl.ds(r * ROW_PITCH + STREAM_LEN, TAIL_LEN)],
                sem,
            )
            return c

        lax.fori_loop(0, GROUP, issue, 0)

    def drain(idx_v, rows_v, sem):
        def body(r, c):
            pltpu.make_async_copy(
                p_hbm.at[idx_v.at[r, pl.ds(0, STREAM_LEN)]],
                rows_v.at[pl.ds(r * ROW_PITCH, STREAM_LEN)],
                sem,
            ).wait()
            pltpu.make_async_copy(
                p_hbm.at[idx_v.at[r, pl.ds(STREAM_LEN, TAIL_LEN)]],
                rows_v.at[pl.ds(r * ROW_PITCH + STREAM_LEN, TAIL_LEN)],
                sem,
            ).wait()
            return c

        lax.fori_loop(0, GROUP, body, 0)

    def accumulate(g, rows_v):
        def row_sum(row0):
            def chunk(kb, accs):
                a0, a1 = accs
                b = row0 + kb * 40
                for u in range(5):
                    r0 = row_pat + (b + 8 * u)
                    r1 = row_pat + (b + 8 * u + 4)
                    a0 = a0 + plsc.load_gather(rows_v, [r0, col_pat])
                    a1 = a1 + plsc.load_gather(rows_v, [r1, col_pat])
                return (a0, a1)

            zero = jnp.zeros((LANES,), jnp.float32)
            a0, a1 = lax.fori_loop(0, SEQ // 40, chunk, (zero, zero))
            acc = a0 + a1
            fold_v[...] = acc
            acc = acc + plsc.load_gather(fold_v, [perm8])
            fold_v[...] = acc
            acc = acc + plsc.load_gather(fold_v, [perm4])
            return acc

        def quad_body(q, c):
            accs = [row_sum((q * 4 + rr) * ROW_PITCH + 1) for rr in range(4)]
            out16 = jnp.where(m0, accs[0],
                              jnp.where(m1, accs[1],
                                        jnp.where(m2, accs[2], accs[3])))
            out_v[pl.ds(g * (GROUP * NUM_CLASSES) + q * LANES, LANES)] = out16
            return c

        lax.fori_loop(0, GROUP // 4, quad_body, 0)

    fire(0, idx_a, rows_a, sem_a)

    def pair_body(gp, carry):
        g = gp * 2
        fire(g + 1, idx_b, rows_b, sem_b)
        drain(idx_a, rows_a, sem_a)
        accumulate(g, rows_a)

        @pl.when(gp < NGROUPS // 2 - 1)
        def _():
            fire(g + 2, idx_a, rows_a, sem_a)

        drain(idx_b, rows_b, sem_b)
        accumulate(g + 1, rows_b)
        return carry

    lax.fori_loop(0, NGROUPS // 2, pair_body, 0)
    pltpu.sync_copy(
        out_v,
        out_hbm.at[pl.ds(wid * (ROWS_PER_W * NUM_CLASSES),
                         ROWS_PER_W * NUM_CLASSES)],
    )


def kernel(x_batch, emb_table, fc_w, fc_b):
    text = x_batch.astype(jnp.int32)
    scale = jnp.float32(1.0 / SEQ)
    w16 = jnp.tile(fc_w.T * scale, (1, 4))
    b16 = jnp.tile(fc_b * scale, 4).reshape(1, LANES)
    p16 = _project(emb_table, w16, b16)
    out = _pool_kernel(text, p16)
    return out.reshape(BATCH, NUM_CLASSES)


# speedup vs baseline: 1.4702x; 1.0989x over previous
import functools

import jax
import jax.numpy as jnp
from jax import lax
from jax.experimental import pallas as pl
from jax.experimental.pallas import tpu as pltpu
from jax.experimental.pallas import tpu_sc as plsc

VOCAB = 100000
EMBED_DIM = 64
NUM_CLASSES = 4
BATCH = 16384
SEQ = 200

LANES = 16
NUM_CORES = 2
NUM_SUBCORES = 16
NUM_WORKERS = NUM_CORES * NUM_SUBCORES
ROWS_PER_W = BATCH // NUM_WORKERS
GROUP = 16
NGROUPS = ROWS_PER_W // GROUP
ROW_PITCH = 208
STREAM_LEN = 128
TAIL_LEN = SEQ + 1 - STREAM_LEN

_PROJ_BLOCK = 10000


def _project_body(e_ref, w_ref, b_ref, o_ref):
    o_ref[...] = (
        jnp.dot(e_ref[...], w_ref[...], preferred_element_type=jnp.float32)
        + b_ref[...]
    )


def _project(emb_table, w4, b4):
    grid = (VOCAB // _PROJ_BLOCK,)
    return pl.pallas_call(
        _project_body,
        grid=grid,
        in_specs=[
            pl.BlockSpec((_PROJ_BLOCK, EMBED_DIM), lambda i: (i, 0)),
            pl.BlockSpec((EMBED_DIM, NUM_CLASSES), lambda i: (0, 0)),
            pl.BlockSpec((1, NUM_CLASSES), lambda i: (0, 0)),
        ],
        out_specs=pl.BlockSpec((_PROJ_BLOCK, NUM_CLASSES), lambda i: (i, 0)),
        out_shape=jax.ShapeDtypeStruct((VOCAB, NUM_CLASSES), jnp.float32),
    )(emb_table, w4, b4)


_sc_mesh = plsc.VectorSubcoreMesh(core_axis_name="c", subcore_axis_name="s")


@functools.partial(
    pl.kernel,
    mesh=_sc_mesh,
    compiler_params=pltpu.CompilerParams(
        use_tc_tiling_on_sc=False, needs_layout_passes=False
    ),
    out_type=jax.ShapeDtypeStruct((BATCH * NUM_CLASSES,), jnp.float32),
    scratch_types=[
        pltpu.VMEM((GROUP, SEQ + 1), jnp.int32),
        pltpu.VMEM((GROUP, SEQ + 1), jnp.int32),
        pltpu.VMEM((GROUP * ROW_PITCH, NUM_CLASSES), jnp.float32),
        pltpu.VMEM((GROUP * ROW_PITCH, NUM_CLASSES), jnp.float32),
        pltpu.VMEM((ROWS_PER_W * NUM_CLASSES,), jnp.float32),
        pltpu.VMEM((LANES,), jnp.float32),
        pltpu.VMEM_SHARED((VOCAB, NUM_CLASSES), jnp.float32),
        pltpu.SemaphoreType.DMA,
        pltpu.SemaphoreType.DMA,
    ],
)
def _pool_kernel(text_hbm, p_hbm, out_hbm, idx_a, idx_b, rows_a, rows_b,
                 out_v, fold_v, p_sh, sem_a, sem_b):
    wid = lax.axis_index("s") * NUM_CORES + lax.axis_index("c")
    row_base = wid * ROWS_PER_W

    lane = lax.iota(jnp.int32, LANES)
    m0 = lane < 4
    m1 = lane < 8
    m2 = lane < 12
    row_pat = lane >> 2
    col_pat = lane & 3
    perm8 = (lane + 8) & 15
    perm4 = (lane + 4) & 15

    def fire(g, idx_v, rows_v, sem):
        pltpu.sync_copy(
            text_hbm.at[pl.ds(row_base + g * GROUP, GROUP)],
            idx_v,
        )

        def issue(r, c):
            pltpu.async_copy(
                p_sh.at[idx_v.at[r, pl.ds(0, STREAM_LEN)]],
                rows_v.at[pl.ds(r * ROW_PITCH, STREAM_LEN)],
                sem,
            )
            pltpu.async_copy(
                p_sh.at[idx_v.at[r, pl.ds(STREAM_LEN, TAIL_LEN)]],
                rows_v.at[pl.ds(r * ROW_PITCH + STREAM_LEN, TAIL_LEN)],
                sem,
            )
            return c

        lax.fori_loop(0, GROUP, issue, 0)

    def drain(idx_v, rows_v, sem):
        def body(r, c):
            pltpu.make_async_copy(
                p_sh.at[idx_v.at[r, pl.ds(0, STREAM_LEN)]],
                rows_v.at[pl.ds(r * ROW_PITCH, STREAM_LEN)],
                sem,
            ).wait()
            pltpu.make_async_copy(
                p_sh.at[idx_v.at[r, pl.ds(STREAM_LEN, TAIL_LEN)]],
                rows_v.at[pl.ds(r * ROW_PITCH + STREAM_LEN, TAIL_LEN)],
                sem,
            ).wait()
            return c

        lax.fori_loop(0, GROUP, body, 0)

    def accumulate(g, rows_v):
        def row_sum(row0):
            def chunk(kb, accs):
                a0, a1 = accs
                b = row0 + kb * 40
                for u in range(5):
                    r0 = row_pat + (b + 8 * u)
                    r1 = row_pat + (b + 8 * u + 4)
                    a0 = a0 + plsc.load_gather(rows_v, [r0, col_pat])
                    a1 = a1 + plsc.load_gather(rows_v, [r1, col_pat])
                return (a0, a1)

            zero = jnp.zeros((LANES,), jnp.float32)
            a0, a1 = lax.fori_loop(0, SEQ // 40, chunk, (zero, zero))
            acc = a0 + a1
            fold_v[...] = acc
            acc = acc + plsc.load_gather(fold_v, [perm8])
            fold_v[...] = acc
            acc = acc + plsc.load_gather(fold_v, [perm4])
            return acc

        def quad_body(q, c):
            accs = [row_sum((q * 4 + rr) * ROW_PITCH + 1) for rr in range(4)]
            out16 = jnp.where(m0, accs[0],
                              jnp.where(m1, accs[1],
                                        jnp.where(m2, accs[2], accs[3])))
            out_v[pl.ds(g * (GROUP * NUM_CLASSES) + q * LANES, LANES)] = out16
            return c

        lax.fori_loop(0, GROUP // 4, quad_body, 0)

    sid = lax.axis_index("s")
    pltpu.sync_copy(
        p_hbm.at[pl.ds(sid * (VOCAB // NUM_SUBCORES), VOCAB // NUM_SUBCORES)],
        p_sh.at[pl.ds(sid * (VOCAB // NUM_SUBCORES), VOCAB // NUM_SUBCORES)],
    )
    plsc.subcore_barrier()

    fire(0, idx_a, rows_a, sem_a)

    def pair_body(gp, carry):
        g = gp * 2
        fire(g + 1, idx_b, rows_b, sem_b)
        drain(idx_a, rows_a, sem_a)
        accumulate(g, rows_a)

        @pl.when(gp < NGROUPS // 2 - 1)
        def _():
            fire(g + 2, idx_a, rows_a, sem_a)

        drain(idx_b, rows_b, sem_b)
        accumulate(g + 1, rows_b)
        return carry

    lax.fori_loop(0, NGROUPS // 2, pair_body, 0)
    pltpu.sync_copy(
        out_v,
        out_hbm.at[pl.ds(wid * (ROWS_PER_W * NUM_CLASSES),
                         ROWS_PER_W * NUM_CLASSES)],
    )


def kernel(x_batch, emb_table, fc_w, fc_b):
    text = x_batch.astype(jnp.int32)
    scale = jnp.float32(1.0 / SEQ)
    w4 = fc_w.T * scale
    b4 = (fc_b * scale).reshape(1, NUM_CLASSES)
    p4 = _project(emb_table, w4, b4)
    out = _pool_kernel(text, p4)
    return out.reshape(BATCH, NUM_CLASSES)
